# baseline (speedup 1.0000x reference)
import functools

import jax
import jax.numpy as jnp
from jax.experimental import pallas as pl
from jax.experimental.pallas import tpu as pltpu

_C = 256
_F = 512


def _rpn_level_kernel(*refs, TH, W, alias):
    if alias:
        (prev_ref, cent_ref, next_ref, w_ref, wcat_ref, bsh_ref, bcat_ref,
         _inL, _inP, _inD, logits_ref, probs_ref, deltas_ref) = refs
    else:
        (prev_ref, cent_ref, next_ref, w_ref, wcat_ref, bsh_ref, bcat_ref,
         logits_ref, probs_ref, deltas_ref) = refs

    prev = prev_ref[0]
    cent = cent_ref[0]
    nxt = next_ref[0]
    ext = jnp.concatenate([prev, cent, nxt], axis=0)

    cols = [ext[dy:dy + TH, dx:dx + W, :].reshape(TH * W, _C)
            for dy in range(3) for dx in range(3)]
    x = jnp.concatenate(cols, axis=1)
    acc = jnp.dot(x, w_ref[...], preferred_element_type=jnp.float32)
    shared = jnp.maximum(acc + bsh_ref[...], 0.0).astype(jnp.bfloat16)

    yt = jax.lax.dot_general(
        wcat_ref[...], shared, (((1,), (1,)), ((), ())),
        preferred_element_type=jnp.float32) + bcat_ref[...]
    logits_t = yt[0:6]
    deltas_t = yt[6:18]

    i = jax.lax.broadcasted_iota(jnp.int32, (6, 6), 0)
    j = jax.lax.broadcasted_iota(jnp.int32, (6, 6), 1)
    perm = ((i ^ 1) == j).astype(jnp.float32)
    swapped = jnp.dot(perm, logits_t, preferred_element_type=jnp.float32)
    m = jnp.maximum(logits_t, swapped)
    e = jnp.exp(logits_t - m)
    esw = jnp.exp(swapped - m)
    probs_t = e / (e + esw)

    logits_ref[0] = logits_t
    probs_ref[0] = probs_t
    deltas_ref[0] = deltas_t


def _run_level(feat, wsh, wcat, bsh, bcat, TH, base, hw_total, carry):
    B, H, W, C = feat.shape
    n_tiles = H // TH
    R = TH * W
    base_blk = base // R
    xp = jnp.pad(feat.astype(jnp.bfloat16),
                 ((0, 0), (TH, TH), (1, 1), (0, 0)))

    alias = carry is not None
    grid = (B, n_tiles)
    kfn = functools.partial(_rpn_level_kernel, TH=TH, W=W, alias=alias)

    in_specs = [
        pl.BlockSpec((1, 1, W + 2, C),
                     lambda b, i: (b, (i + 1) * TH - 1, 0, 0)),
        pl.BlockSpec((1, TH, W + 2, C), lambda b, i: (b, i + 1, 0, 0)),
        pl.BlockSpec((1, 1, W + 2, C),
                     lambda b, i: (b, (i + 2) * TH, 0, 0)),
        pl.BlockSpec((9 * _C, _F), lambda b, i: (0, 0)),
        pl.BlockSpec((18, _F), lambda b, i: (0, 0)),
        pl.BlockSpec((1, _F), lambda b, i: (0, 0)),
        pl.BlockSpec((18, 1), lambda b, i: (0, 0)),
    ]
    kwargs = {}
    if alias:
        in_specs += [pl.BlockSpec(memory_space=pltpu.MemorySpace.HBM)] * 3
        kwargs['input_output_aliases'] = {7: 0, 8: 1, 9: 2}

    def omap(b, i, bb=base_blk):
        return (b, 0, bb + i)

    logits, probs, deltas = pl.pallas_call(
        kfn,
        grid=grid,
        in_specs=in_specs,
        out_specs=[
            pl.BlockSpec((1, 6, R), omap),
            pl.BlockSpec((1, 6, R), omap),
            pl.BlockSpec((1, 12, R), omap),
        ],
        out_shape=[
            jax.ShapeDtypeStruct((B, 6, hw_total), jnp.float32),
            jax.ShapeDtypeStruct((B, 6, hw_total), jnp.float32),
            jax.ShapeDtypeStruct((B, 12, hw_total), jnp.float32),
        ],
        compiler_params=pltpu.CompilerParams(
            dimension_semantics=("parallel", "parallel")),
        **kwargs,
    )(xp, xp, xp, wsh, wcat, bsh, bcat, *(carry if alias else ()))
    return logits, probs, deltas


def kernel(feat_p2, feat_p3, feat_p4, feat_p5, feat_p6,
           w_shared, b_shared, w_cls, b_cls, w_delta, b_delta):
    feats = [feat_p2, feat_p3, feat_p4, feat_p5, feat_p6]
    tile_h = {256: 8, 128: 16, 64: 32, 32: 32, 16: 16}
    B = feat_p2.shape[0]
    hw_total = sum(f.shape[1] * f.shape[2] for f in feats)

    wsh = w_shared.astype(jnp.bfloat16).reshape(9 * _C, _F)
    wcat = jnp.concatenate([w_cls[0, 0], w_delta[0, 0]],
                           axis=1).T.astype(jnp.bfloat16)
    bsh = b_shared.reshape(1, _F)
    bcat = jnp.concatenate([b_cls, b_delta]).reshape(18, 1)

    carry = None
    base = 0
    for feat in feats:
        _, H, W, _ = feat.shape
        carry = _run_level(feat, wsh, wcat, bsh, bcat, tile_h[H],
                           base, hw_total, carry)
        base += H * W

    lg_t, pr_t, dl_t = carry
    lg = lg_t.reshape(B, 3, 2, hw_total).transpose(0, 3, 1, 2)
    pr = pr_t.reshape(B, 3, 2, hw_total).transpose(0, 3, 1, 2)
    dl = dl_t.reshape(B, 3, 4, hw_total).transpose(0, 3, 1, 2)
    return (lg.reshape(B, hw_total * 3, 2),
            pr.reshape(B, hw_total * 3, 2),
            dl.reshape(B, hw_total * 3, 4))


# speedup vs baseline: 2.0776x; 2.0776x over previous
import functools

import jax
import jax.numpy as jnp
from jax.experimental import pallas as pl
from jax.experimental.pallas import tpu as pltpu

_C = 256
_F = 512


def _rpn_level_kernel(*refs, TH, W, alias):
    if alias:
        (prev_ref, cent_ref, next_ref, w_ref, wcat_ref, bsh_ref, bcat_ref,
         _packed_in, packed_ref) = refs
    else:
        (prev_ref, cent_ref, next_ref, w_ref, wcat_ref, bsh_ref, bcat_ref,
         packed_ref) = refs

    prev = prev_ref[0]
    cent = cent_ref[0]
    nxt = next_ref[0]
    ext = jnp.concatenate([prev, cent, nxt], axis=0)

    cols = [ext[dy:dy + TH, dx:dx + W, :].reshape(TH * W, _C)
            for dy in range(3) for dx in range(3)]
    x = jnp.concatenate(cols, axis=1)
    acc = jnp.dot(x, w_ref[...], preferred_element_type=jnp.float32)
    shared = jnp.maximum(acc + bsh_ref[...], 0.0).astype(jnp.bfloat16)

    y = jnp.dot(shared, wcat_ref[...],
                preferred_element_type=jnp.float32) + bcat_ref[...]
    logits = y[:, 0:6]
    deltas = y[:, 6:18]

    i = jax.lax.broadcasted_iota(jnp.int32, (6, 6), 0)
    j = jax.lax.broadcasted_iota(jnp.int32, (6, 6), 1)
    perm = ((i ^ 1) == j).astype(jnp.float32)
    swapped = jnp.dot(logits, perm, preferred_element_type=jnp.float32)
    m = jnp.maximum(logits, swapped)
    e = jnp.exp(logits - m)
    esw = jnp.exp(swapped - m)
    probs = e / (e + esw)

    for a in range(3):
        packed_ref[0, a::3, :] = jnp.concatenate(
            [logits[:, 2 * a:2 * a + 2], probs[:, 2 * a:2 * a + 2],
             deltas[:, 4 * a:4 * a + 4]], axis=1)


def _run_level(feat, wsh, wcat, bsh, bcat, TH, base, n_total, carry):
    B, H, W, C = feat.shape
    n_tiles = H // TH
    R = TH * W * 3
    base_blk = base // R
    xp = jnp.pad(feat.astype(jnp.bfloat16),
                 ((0, 0), (TH, TH), (1, 1), (0, 0)))

    alias = carry is not None
    grid = (B, n_tiles)
    kfn = functools.partial(_rpn_level_kernel, TH=TH, W=W, alias=alias)

    in_specs = [
        pl.BlockSpec((1, 1, W + 2, C),
                     lambda b, i: (b, (i + 1) * TH - 1, 0, 0)),
        pl.BlockSpec((1, TH, W + 2, C), lambda b, i: (b, i + 1, 0, 0)),
        pl.BlockSpec((1, 1, W + 2, C),
                     lambda b, i: (b, (i + 2) * TH, 0, 0)),
        pl.BlockSpec((9 * _C, _F), lambda b, i: (0, 0)),
        pl.BlockSpec((_F, 18), lambda b, i: (0, 0)),
        pl.BlockSpec((1, _F), lambda b, i: (0, 0)),
        pl.BlockSpec((1, 18), lambda b, i: (0, 0)),
    ]
    kwargs = {}
    if alias:
        in_specs += [pl.BlockSpec(memory_space=pltpu.MemorySpace.HBM)]
        kwargs['input_output_aliases'] = {7: 0}

    def omap(b, i, bb=base_blk):
        return (b, bb + i, 0)

    packed = pl.pallas_call(
        kfn,
        grid=grid,
        in_specs=in_specs,
        out_specs=pl.BlockSpec((1, R, 8), omap),
        out_shape=jax.ShapeDtypeStruct((B, n_total, 8), jnp.float32),
        compiler_params=pltpu.CompilerParams(
            dimension_semantics=("parallel", "parallel")),
        **kwargs,
    )(xp, xp, xp, wsh, wcat, bsh, bcat,
      *((carry,) if alias else ()))
    return packed


def kernel(feat_p2, feat_p3, feat_p4, feat_p5, feat_p6,
           w_shared, b_shared, w_cls, b_cls, w_delta, b_delta):
    feats = [feat_p2, feat_p3, feat_p4, feat_p5, feat_p6]
    tile_h = {256: 8, 128: 16, 64: 32, 32: 32, 16: 16}
    n_total = 3 * sum(f.shape[1] * f.shape[2] for f in feats)

    wsh = w_shared.astype(jnp.bfloat16).reshape(9 * _C, _F)
    wcat = jnp.concatenate([w_cls[0, 0], w_delta[0, 0]],
                           axis=1).astype(jnp.bfloat16)
    bsh = b_shared.reshape(1, _F)
    bcat = jnp.concatenate([b_cls, b_delta]).reshape(1, 18)

    carry = None
    base = 0
    for feat in feats:
        _, H, W, _ = feat.shape
        carry = _run_level(feat, wsh, wcat, bsh, bcat, tile_h[H],
                           base, n_total, carry)
        base += H * W * 3

    return (carry[..., 0:2], carry[..., 2:4], carry[..., 4:8])


# speedup vs baseline: 3.0980x; 1.4911x over previous
import functools

import jax
import jax.numpy as jnp
from jax.experimental import pallas as pl
from jax.experimental.pallas import tpu as pltpu

_C = 256
_F = 512


def _rpn_level_kernel(*refs, TH, W, n_tiles, alias):
    if alias:
        (prev_ref, cent_ref, next_ref, w_ref, wcat_ref, bsh_ref, bcat_ref,
         _inL, _inP, _inD, logits_ref, probs_ref, deltas_ref) = refs
    else:
        (prev_ref, cent_ref, next_ref, w_ref, wcat_ref, bsh_ref, bcat_ref,
         logits_ref, probs_ref, deltas_ref) = refs

    i = pl.program_id(1)
    prev = jnp.where(i == 0, 0.0, prev_ref[0]).astype(jnp.bfloat16)
    nxt = jnp.where(i == n_tiles - 1, 0.0,
                    next_ref[0]).astype(jnp.bfloat16)
    cent = cent_ref[0].astype(jnp.bfloat16)
    rows = jnp.concatenate([prev, cent, nxt], axis=0)
    zcol = jnp.zeros((TH + 2, 1, _C), jnp.bfloat16)
    ext = jnp.concatenate([zcol, rows, zcol], axis=1)

    cols = [ext[dy:dy + TH, dx:dx + W, :].reshape(TH * W, _C)
            for dy in range(3) for dx in range(3)]
    x = jnp.concatenate(cols, axis=1)
    acc = jnp.dot(x, w_ref[...], preferred_element_type=jnp.float32)
    shared = jnp.maximum(acc + bsh_ref[...], 0.0).astype(jnp.bfloat16)

    y = jnp.dot(shared, wcat_ref[...],
                preferred_element_type=jnp.float32) + bcat_ref[...]
    logits = y[:, 0:6]
    deltas = y[:, 6:18]

    ii = jax.lax.broadcasted_iota(jnp.int32, (6, 6), 0)
    jj = jax.lax.broadcasted_iota(jnp.int32, (6, 6), 1)
    perm = ((ii ^ 1) == jj).astype(jnp.float32)
    swapped = jnp.dot(logits, perm, preferred_element_type=jnp.float32)
    m = jnp.maximum(logits, swapped)
    e = jnp.exp(logits - m)
    esw = jnp.exp(swapped - m)
    probs = e / (e + esw)

    for a in range(3):
        logits_ref[0, a::3, :] = logits[:, 2 * a:2 * a + 2]
        probs_ref[0, a::3, :] = probs[:, 2 * a:2 * a + 2]
        deltas_ref[0, a::3, :] = deltas[:, 4 * a:4 * a + 4]


def _run_level(feat, wsh, wcat, bsh, bcat, TH, base, n_total, carry):
    B, H, W, C = feat.shape
    n_tiles = H // TH
    R = TH * W * 3
    base_blk = base // R

    alias = carry is not None
    grid = (B, n_tiles)
    kfn = functools.partial(_rpn_level_kernel, TH=TH, W=W,
                            n_tiles=n_tiles, alias=alias)

    in_specs = [
        pl.BlockSpec((1, 1, W, C),
                     lambda b, i: (b, jnp.maximum(i * TH - 1, 0), 0, 0)),
        pl.BlockSpec((1, TH, W, C), lambda b, i: (b, i, 0, 0)),
        pl.BlockSpec((1, 1, W, C),
                     lambda b, i: (b, jnp.minimum((i + 1) * TH, H - 1),
                                   0, 0)),
        pl.BlockSpec((9 * _C, _F), lambda b, i: (0, 0)),
        pl.BlockSpec((_F, 18), lambda b, i: (0, 0)),
        pl.BlockSpec((1, _F), lambda b, i: (0, 0)),
        pl.BlockSpec((1, 18), lambda b, i: (0, 0)),
    ]
    kwargs = {}
    if alias:
        in_specs += [pl.BlockSpec(memory_space=pltpu.MemorySpace.HBM)] * 3
        kwargs['input_output_aliases'] = {7: 0, 8: 1, 9: 2}

    def omap(b, i, bb=base_blk):
        return (b, bb + i, 0)

    outs = pl.pallas_call(
        kfn,
        grid=grid,
        in_specs=in_specs,
        out_specs=[
            pl.BlockSpec((1, R, 2), omap),
            pl.BlockSpec((1, R, 2), omap),
            pl.BlockSpec((1, R, 4), omap),
        ],
        out_shape=[
            jax.ShapeDtypeStruct((B, n_total, 2), jnp.float32),
            jax.ShapeDtypeStruct((B, n_total, 2), jnp.float32),
            jax.ShapeDtypeStruct((B, n_total, 4), jnp.float32),
        ],
        compiler_params=pltpu.CompilerParams(
            dimension_semantics=("parallel", "parallel")),
        **kwargs,
    )(feat, feat, feat, wsh, wcat, bsh, bcat, *(carry if alias else ()))
    return outs


def kernel(feat_p2, feat_p3, feat_p4, feat_p5, feat_p6,
           w_shared, b_shared, w_cls, b_cls, w_delta, b_delta):
    feats = [feat_p2, feat_p3, feat_p4, feat_p5, feat_p6]
    tile_h = {256: 8, 128: 16, 64: 32, 32: 32, 16: 16}
    n_total = 3 * sum(f.shape[1] * f.shape[2] for f in feats)

    wsh = w_shared.astype(jnp.bfloat16).reshape(9 * _C, _F)
    wcat = jnp.concatenate([w_cls[0, 0], w_delta[0, 0]],
                           axis=1).astype(jnp.bfloat16)
    bsh = b_shared.reshape(1, _F)
    bcat = jnp.concatenate([b_cls, b_delta]).reshape(1, 18)

    carry = None
    base = 0
    for feat in feats:
        _, H, W, _ = feat.shape
        carry = _run_level(feat, wsh, wcat, bsh, bcat, tile_h[H],
                           base, n_total, carry)
        base += H * W * 3

    return carry
